# Initial kernel scaffold; baseline (speedup 1.0000x reference)
import jax
import jax.numpy as jnp
from jax.experimental import pallas as pl


def kernel(x, edge_index, edge_attr, batch, params):
    raise NotImplementedError("write your pallas kernel here")


# speedup vs baseline: 3.7674x; 3.7674x over previous
import functools

import jax
import jax.numpy as jnp
from jax import lax
from jax.experimental import pallas as pl
from jax.experimental.pallas import tpu as pltpu
from jax.experimental.pallas import tpu_sc as plsc

F32 = jnp.float32

KN = 10000
KE = 320000
KDIN = 128
KDE = 16
KH = 32
KHF = 64
KL = 4
KG = 4
KNG = 64

NC = 2
NS = 16
NW = NC * NS
EPW = KE // NW
CH = 80
NCHUNK = EPW // CH
NPAD = 10240
RPT = NPAD // NS


def _make_edge_pass(with_deg):
  mesh = plsc.VectorSubcoreMesh(core_axis_name="c", subcore_axis_name="s")
  out_type = [jax.ShapeDtypeStruct((NC, NPAD, 4 * KH), F32)]
  scratch = [
      pltpu.VMEM_SHARED((NPAD, 4 * KH), F32),
      pltpu.VMEM((CH,), jnp.int32),
      pltpu.VMEM((CH,), jnp.int32),
      pltpu.VMEM((CH, 4 * KH), F32),
      pltpu.VMEM((CH, 4 * KH), F32),
      pltpu.VMEM((CH, KH), F32),
      pltpu.VMEM((CH, 4 * KH), F32),
      pltpu.SemaphoreType.DMA,
      pltpu.SemaphoreType.DMA,
      pltpu.SemaphoreType.DMA,
  ]
  if with_deg:
    out_type.append(jax.ShapeDtypeStruct((NC * NPAD,), F32))
    scratch += [
        pltpu.VMEM_SHARED((NPAD,), F32),
        pltpu.VMEM((CH,), F32),
        pltpu.VMEM((CH,), F32),
    ]

  def body(tab, ef, sidx_hbm, didx_hbm, *rest):
    if with_deg:
      (out, degout, acc, sidx_v, didx_v, srows, drows, efc, arows,
       sem0, sem1, sem2, dacc, dstage, ones_v) = rest
    else:
      (out, acc, sidx_v, didx_v, srows, drows, efc, arows,
       sem0, sem1, sem2) = rest
    c = lax.axis_index("c")
    s = lax.axis_index("s")
    wid = s * NC + c
    z16 = jnp.zeros((16,), F32)
    r0 = s * RPT

    def zrow(j, carry):
      for k in range(4 * KH // 16):
        arows[j, pl.ds(16 * k, 16)] = z16
      return carry
    lax.fori_loop(0, CH, zrow, 0, unroll=4)

    def zacc(j, carry):
      pltpu.sync_copy(arows,
                      acc.at[pl.ds(pl.multiple_of(r0 + j * CH, 8), CH)])
      return carry
    lax.fori_loop(0, RPT // CH, zacc, 0)
    if with_deg:
      for k in range(CH // 16):
        dstage[pl.ds(16 * k, 16)] = z16
      def zdacc(j, carry):
        pltpu.sync_copy(dstage,
                        dacc.at[pl.ds(pl.multiple_of(r0 + j * CH, 8), CH)])
        return carry
      lax.fori_loop(0, RPT // CH, zdacc, 0)
      one16 = jnp.ones((16,), F32)
      for k in range(CH // 16):
        ones_v[pl.ds(16 * k, 16)] = one16
    plsc.subcore_barrier()

    ebase = wid * EPW

    def chunk(i, carry):
      base = pl.multiple_of(ebase + i * CH, 8)
      pltpu.sync_copy(sidx_hbm.at[pl.ds(base, CH)], sidx_v)
      pltpu.sync_copy(didx_hbm.at[pl.ds(base, CH)], didx_v)
      g0 = pltpu.async_copy(tab.at[sidx_v], srows, sem0)
      g1 = pltpu.async_copy(tab.at[didx_v], drows, sem1)
      g2 = pltpu.async_copy(ef.at[pl.ds(base, CH)], efc, sem2)
      g0.wait()
      g1.wait()
      g2.wait()

      def row(j, rcarry):
        for h in range(KH // 16):
          arows[j, pl.ds(16 * h, 16)] = srows[j, pl.ds(16 * h, 16)]
          a = drows[j, pl.ds(2 * KH + 16 * h, 16)]
          b = srows[j, pl.ds(KH + 16 * h, 16)]
          e = efc[j, pl.ds(16 * h, 16)]
          arows[j, pl.ds(KH + 16 * h, 16)] = jnp.maximum(a + b + e, 0.0)
        return rcarry
      lax.fori_loop(0, CH, row, 0, unroll=4)
      pltpu.sync_copy(arows, acc.at[didx_v], add=True)
      if with_deg:
        pltpu.sync_copy(ones_v, dacc.at[didx_v], add=True)
      return carry
    lax.fori_loop(0, NCHUNK, chunk, 0)

    plsc.subcore_barrier()
    pltpu.sync_copy(acc.at[pl.ds(r0, RPT)], out.at[c, pl.ds(r0, RPT)])
    if with_deg:
      pltpu.sync_copy(dacc.at[pl.ds(r0, RPT)],
                      degout.at[pl.ds(c * NPAD + r0, RPT)])

  return pl.kernel(body, out_type=out_type, mesh=mesh, scratch_types=scratch)


_edge_pass_deg = _make_edge_pass(True)
_edge_pass = _make_edge_pass(False)


def _graph_onehot_t(brow):
  gids = lax.broadcasted_iota(jnp.int32, (KNG, 1), 0)
  pt = (brow == gids).astype(F32)
  cnt = jnp.maximum(jnp.sum(pt, axis=1, keepdims=True), 1.0)
  return pt, cnt


def _mm(a, b):
  return jnp.dot(a, b, preferred_element_type=F32)


def _tc0_body(x_ref, brow_ref, wi_ref, bi_ref, wne_ref, bne_ref, wnl_ref,
              bnl_ref, msgw_ref, msgb_ref, bdi_ref, bdj_ref,
              out0_ref, u0_ref, st_ref):
  x = x_ref[...]
  pt, cnt = _graph_onehot_t(brow_ref[...])
  g = _mm(pt, x) / cnt
  o = _mm(g, wne_ref[...]) + bne_ref[...]
  o = o + jnp.maximum(_mm(o, wnl_ref[...]) + bnl_ref[...], 0.0)
  out0_ref[...] = o
  u0 = _mm(x, wi_ref[...]) + bi_ref[...]
  u0_ref[...] = u0
  um = _mm(u0, msgw_ref[...]) + msgb_ref[...]
  u1 = _mm(um, bdi_ref[...])
  u2 = _mm(um, bdj_ref[...])
  st_ref[0:KN, 0:KH] = um
  st_ref[0:KN, KH:2 * KH] = u2
  st_ref[0:KN, 2 * KH:3 * KH] = u1
  st_ref[0:KN, 3 * KH:4 * KH] = jnp.zeros((KN, KH), F32)
  st_ref[KN:NPAD, :] = jnp.zeros((NPAD - KN, 4 * KH), F32)


_EB = 2000


def _ef_body(ea_ref, w_ref, b_ref, out_ref):
  ea = ea_ref[...]
  for l in range(KL):
    out_ref[l] = _mm(ea, w_ref[l]) + b_ref[l]


def _avg_body(deg_ref, bcol_ref, avg_out):
  degrow = deg_ref[0:1, 0:KN] + deg_ref[1:2, 0:KN]
  pcol = (bcol_ref[...] ==
          lax.broadcasted_iota(jnp.int32, (1, KNG), 1)).astype(F32)
  eg = _mm(degrow, pcol)
  cnt = jnp.maximum(jnp.sum(pcol, axis=0, keepdims=True), 1.0)
  avg_g = eg / cnt
  avgn = lax.dot_general(pcol, avg_g, (((1,), (1,)), ((), ())),
                         preferred_element_type=F32)
  avg_out[...] = jnp.maximum(avgn, 1.0)


def _nodeupd_body(u_ref, st_ref, s01_ref, avg_ref, bdo_ref, w1a_ref,
                  w1b_ref, b1_ref, w2_ref, b2_ref, un_out):
  um = st_ref[:, 0:KH]
  s0 = s01_ref[0, :, 0:KH] + s01_ref[1, :, 0:KH]
  s1 = s01_ref[0, :, KH:2 * KH] + s01_ref[1, :, KH:2 * KH]
  aggr = s0 + _mm(s1, bdo_ref[...])
  up1 = _mm(um, w1a_ref[...]) + _mm(aggr, w1b_ref[...]) + b1_ref[...]
  up2 = up1 + _mm(up1, w2_ref[...]) + b2_ref[...]
  un_out[...] = u_ref[...] + (up2 + um) / avg_ref[...]


_NB = 2000


def _make_graph_body(l):
  def body(*refs):
    it = iter(refs)
    un_ref = next(it)
    brow_ref = next(it)
    out0_ref = next(it)
    extw_ref = next(it)
    extb_ref = next(it)
    extlw_ref = next(it)
    extlb_ref = next(it)
    if l < KL - 1:
      gam_ref = next(it)
      bet_ref = next(it)
      msgw_ref = next(it)
      msgb_ref = next(it)
      bdi_ref = next(it)
      bdj_ref = next(it)
      un_out = next(it)
      out0_out = next(it)
      st_out = next(it)
    else:
      aftw_ref = next(it)
      aftb_ref = next(it)
      finw_ref = next(it)
      finb_ref = next(it)
      res_out = next(it)

    un = un_ref[...]
    pt, cnt = _graph_onehot_t(brow_ref[...])
    gm = _mm(pt, un) / cnt
    ge = _mm(gm, extw_ref[...]) + extb_ref[...]
    ge = ge + jnp.maximum(_mm(ge, extlw_ref[...]) + extlb_ref[...], 0.0)
    o = out0_ref[...] + ge * (1.0 / KL)
    if l < KL - 1:
      out0_out[...] = o
      mu = jnp.mean(un, axis=0, keepdims=True)
      var = jnp.mean((un - mu) ** 2, axis=0, keepdims=True)
      ub = (un - mu) * lax.rsqrt(var + 1e-5) * gam_ref[...] + bet_ref[...]
      un_out[...] = ub
      um2 = _mm(ub, msgw_ref[...]) + msgb_ref[...]
      u1n = _mm(um2, bdi_ref[...])
      u2n = _mm(um2, bdj_ref[...])
      st_out[0:KN, 0:KH] = um2
      st_out[0:KN, KH:2 * KH] = u2n
      st_out[0:KN, 2 * KH:3 * KH] = u1n
      st_out[0:KN, 3 * KH:4 * KH] = jnp.zeros((KN, KH), F32)
      st_out[KN:NPAD, :] = jnp.zeros((NPAD - KN, 4 * KH), F32)
    else:
      t = jnp.maximum(_mm(o, aftw_ref[...]) + aftb_ref[...], 0.0) + o
      res_out[...] = _mm(t, finw_ref[...]) + finb_ref[...]
  return body


def _blockdiag(w):
  return jax.scipy.linalg.block_diag(*[w[g] for g in range(KG)])


def kernel(x, edge_index, edge_attr, batch, params):
  p = params
  src = edge_index[0].astype(jnp.int32)
  dst = edge_index[1].astype(jnp.int32)
  brow = batch.reshape(1, KN).astype(jnp.int32)
  bcol = batch.reshape(KN, 1).astype(jnp.int32)

  bdi = [_blockdiag(p['o2i_W'][l]) for l in range(KL)]
  bdj = [_blockdiag(p['o2j_W'][l]) for l in range(KL)]
  bdo = [_blockdiag(p['o2_W'][l]) for l in range(KL)]
  r1 = lambda b: b.reshape(1, -1)

  out0, u0, st = pl.pallas_call(
      _tc0_body,
      out_shape=[
          jax.ShapeDtypeStruct((KNG, KHF), F32),
          jax.ShapeDtypeStruct((KN, KH), F32),
          jax.ShapeDtypeStruct((NPAD, 4 * KH), F32),
      ],
  )(x, brow, p['initial_W'], r1(p['initial_b']),
    p['noprop_ext_W'], r1(p['noprop_ext_b']),
    p['noprop_lin_W'], r1(p['noprop_lin_b']),
    p['msg_W'][0], r1(p['msg_b'][0]), bdi[0], bdj[0])

  ef_all = pl.pallas_call(
      _ef_body,
      grid=(KE // _EB,),
      in_specs=[
          pl.BlockSpec((_EB, KDE), lambda e: (e, 0)),
          pl.BlockSpec((KL, KDE, KH), lambda e: (0, 0, 0)),
          pl.BlockSpec((KL, 1, KH), lambda e: (0, 0, 0)),
      ],
      out_specs=pl.BlockSpec((KL, _EB, KH), lambda e: (0, e, 0)),
      out_shape=jax.ShapeDtypeStruct((KL, KE, KH), F32),
  )(edge_attr, p['edge_W'], p['edge_b'].reshape(KL, 1, KH))

  u_state = u0
  avg = None
  res = None
  for l in range(KL):
    if l == 0:
      s01, deg = _edge_pass_deg(st, ef_all[l], src, dst)
    else:
      (s01,) = _edge_pass(st, ef_all[l], src, dst)

    if l == 0:
      avg = pl.pallas_call(
          _avg_body,
          out_shape=jax.ShapeDtypeStruct((KN, 1), F32),
      )(deg.reshape(NC, NPAD), bcol)

    wspec = lambda shp: pl.BlockSpec(shp, lambda i: (0,) * len(shp))
    un = pl.pallas_call(
        _nodeupd_body,
        grid=(KN // _NB,),
        in_specs=[
            pl.BlockSpec((_NB, KH), lambda i: (i, 0)),
            pl.BlockSpec((_NB, 4 * KH), lambda i: (i, 0)),
            pl.BlockSpec((NC, _NB, 4 * KH), lambda i: (0, i, 0)),
            pl.BlockSpec((_NB, 1), lambda i: (i, 0)),
            wspec((KH, KH)), wspec((KH, KH)), wspec((KH, KH)),
            wspec((1, KH)), wspec((KH, KH)), wspec((1, KH)),
        ],
        out_specs=pl.BlockSpec((_NB, KH), lambda i: (i, 0)),
        out_shape=jax.ShapeDtypeStruct((KN, KH), F32),
    )(u_state, st, s01, avg, bdo[l],
      p['upd1_W'][l][:KH], p['upd1_W'][l][KH:], r1(p['upd1_b'][l]),
      p['upd2_W'][l], r1(p['upd2_b'][l]))

    ins = [un, brow, out0,
           p['ext_W'][l], r1(p['ext_b'][l]),
           p['ext_lin_W'][l], r1(p['ext_lin_b'][l])]
    if l < KL - 1:
      ins += [r1(p['bn_gamma'][l + 1]), r1(p['bn_beta'][l + 1]),
              p['msg_W'][l + 1], r1(p['msg_b'][l + 1]),
              bdi[l + 1], bdj[l + 1]]
      out_shape = [
          jax.ShapeDtypeStruct((KN, KH), F32),
          jax.ShapeDtypeStruct((KNG, KHF), F32),
          jax.ShapeDtypeStruct((NPAD, 4 * KH), F32),
      ]
    else:
      ins += [p['after_W'], r1(p['after_b']),
              p['final_W'], p['final_b'].reshape(1, 1)]
      out_shape = [jax.ShapeDtypeStruct((KNG, 1), F32)]
    outs = pl.pallas_call(_make_graph_body(l), out_shape=out_shape)(*ins)
    if l < KL - 1:
      u_state, out0, st = outs
    else:
      res = outs[0]

  return res.reshape(KNG)
